# baseline (speedup 1.0000x reference)
import functools
import jax
import jax.numpy as jnp
from jax import lax
from jax.experimental import pallas as pl
from jax.experimental.pallas import tpu as pltpu
from jax.experimental.pallas import tpu_sc as plsc

_NC, _NS = 2, 16
_NW = _NC * _NS


def _nodes_kernel(rf_ref, w_ref, b_ref, out_ref):
    rf = rf_ref[0]
    nodes = jax.lax.dot_general(
        rf, w_ref[...], (((1,), (1,)), ((), ())),
        preferred_element_type=jnp.float32)
    out_ref[0] = nodes + b_ref[...]


def _topk_kernel(nodes_ref, out_vals_ref, out_idx_ref, *, R, K, N):
    b = pl.program_id(0)
    rb = pl.program_id(1)
    nodes_all = nodes_ref[0]
    rows = nodes_ref[0, pl.ds(rb * R, R), :]

    sim = jax.lax.dot_general(
        rows, nodes_all, (((1,), (1,)), ((), ())),
        preferred_element_type=jnp.float32)

    col = jax.lax.broadcasted_iota(jnp.int32, (R, N), 1)
    row_g = rb * R + jax.lax.broadcasted_iota(jnp.int32, (R, N), 0)
    sim = jnp.where(col == row_g, jnp.float32(-1e9), sim)

    lane_k = jax.lax.broadcasted_iota(jnp.int32, (R, K), 1)
    vals = jnp.zeros((R, K), jnp.float32)
    idxs = jnp.zeros((R, K), jnp.int32)
    m = jnp.float32(0)
    for kk in range(K):
        work = sim if kk == 0 else jnp.where(sim < m, sim, -jnp.inf)
        m = jnp.max(work, axis=1, keepdims=True)
        vals = jnp.where(lane_k == kk, m, vals)
        first = jnp.min(jnp.where(work == m, col, N), axis=1, keepdims=True)
        idxs = jnp.where(lane_k == kk, first + b * N, idxs)
    out_vals_ref[0] = vals
    out_idx_ref[0] = idxs


def _mean_mlp_kernel(g_ref, nodes_ref, wmsg_ref, bmsg_ref, wout_ref, bout_ref,
                     out_ref, *, R, K, W):
    g = g_ref[0]
    H = nodes_ref.shape[2]
    acc = g[:, 0:H]
    for kk in range(1, K):
        acc = acc + g[:, kk * W:kk * W + H]
    neigh = acc * jnp.float32(1.0 / K)
    rows = nodes_ref[0]
    msgs = jax.lax.dot_general(
        neigh, wmsg_ref[...], (((1,), (1,)), ((), ())),
        preferred_element_type=jnp.float32) + bmsg_ref[...]
    msgs = jnp.maximum(msgs, 0.0)
    updated = rows + msgs
    out = jax.lax.dot_general(
        updated, wout_ref[...], (((1,), (1,)), ((), ())),
        preferred_element_type=jnp.float32) + bout_ref[...]
    out_ref[0] = out


def _sc_gather(table, idx_flat, E_per_w, chunk, H):
    E = idx_flat.shape[0]
    mesh = plsc.VectorSubcoreMesh(core_axis_name="c", subcore_axis_name="s")

    @functools.partial(
        pl.kernel, mesh=mesh,
        out_type=jax.ShapeDtypeStruct((E, H), jnp.float32),
        scratch_types=[
            pltpu.VMEM((chunk,), jnp.int32),
            pltpu.VMEM((chunk, H), jnp.float32),
            pltpu.SemaphoreType.DMA,
        ],
    )
    def k(table_hbm, idx_hbm, out_hbm, idx_v, rows_v, sem):
        wid = lax.axis_index("s") * _NC + lax.axis_index("c")
        base = wid * E_per_w
        for j in range(E_per_w // chunk):
            off = base + j * chunk
            pltpu.sync_copy(idx_hbm.at[pl.ds(off, chunk)], idx_v)
            pltpu.async_copy(table_hbm.at[idx_v], rows_v, sem).wait()
            pltpu.sync_copy(rows_v, out_hbm.at[pl.ds(off, chunk)])

    return k(table, idx_flat)


def kernel(region_features, W_node, b_node, W_msg, b_msg, W_out, b_out):
    B, N, D = region_features.shape
    H = W_node.shape[0]
    K = min(6, N - 1)
    R = 128

    nodes = pl.pallas_call(
        _nodes_kernel,
        grid=(B,),
        in_specs=[
            pl.BlockSpec((1, N, D), lambda b: (b, 0, 0)),
            pl.BlockSpec((H, D), lambda b: (0, 0)),
            pl.BlockSpec((1, H), lambda b: (0, 0)),
        ],
        out_specs=pl.BlockSpec((1, N, H), lambda b: (b, 0, 0)),
        out_shape=jax.ShapeDtypeStruct((B, N, H), jnp.float32),
    )(region_features, W_node, b_node.reshape(1, H))

    vals, idxg = pl.pallas_call(
        functools.partial(_topk_kernel, R=R, K=K, N=N),
        grid=(B, N // R),
        in_specs=[pl.BlockSpec((1, N, H), lambda b, rb: (b, 0, 0))],
        out_specs=[
            pl.BlockSpec((1, R, K), lambda b, rb: (b, rb, 0)),
            pl.BlockSpec((1, R, K), lambda b, rb: (b, rb, 0)),
        ],
        out_shape=[
            jax.ShapeDtypeStruct((B, N, K), jnp.float32),
            jax.ShapeDtypeStruct((B, N, K), jnp.int32),
        ],
    )(nodes)

    E = B * N * K
    E_per_w = E // _NW
    W = 128
    table_pad = jnp.concatenate(
        [nodes.reshape(B * N, H),
         jnp.zeros((B * N, W - H), jnp.float32)], axis=1)
    gathered = _sc_gather(table_pad, idxg.reshape(E),
                          E_per_w, min(512, E_per_w), W)

    out = pl.pallas_call(
        functools.partial(_mean_mlp_kernel, R=R, K=K, W=W),
        grid=(B, N // R),
        in_specs=[
            pl.BlockSpec((1, R, K * W), lambda b, rb: (b, rb, 0)),
            pl.BlockSpec((1, R, H), lambda b, rb: (b, rb, 0)),
            pl.BlockSpec((H, H), lambda b, rb: (0, 0)),
            pl.BlockSpec((1, H), lambda b, rb: (0, 0)),
            pl.BlockSpec((D, H), lambda b, rb: (0, 0)),
            pl.BlockSpec((1, D), lambda b, rb: (0, 0)),
        ],
        out_specs=pl.BlockSpec((1, R, D), lambda b, rb: (b, rb, 0)),
        out_shape=jax.ShapeDtypeStruct((B, N, D), jnp.float32),
    )(gathered.reshape(B, N, K * W), nodes, W_msg, b_msg.reshape(1, H),
      W_out, b_out.reshape(1, D))

    return (out, vals)


# speedup vs baseline: 2.2284x; 2.2284x over previous
import functools
import jax
import jax.numpy as jnp
from jax.experimental import pallas as pl
from jax.experimental.pallas import tpu as pltpu


def _subtile(nodes_vmem, wmsg_ref, bmsg_ref, wout_ref, bout_ref, rbt, R, K, N):
    nodes_all = nodes_vmem[...]
    rows = nodes_vmem[pl.ds(rbt * R, R), :]

    sim = jax.lax.dot_general(
        rows, nodes_all, (((1,), (1,)), ((), ())),
        preferred_element_type=jnp.float32)

    col = jax.lax.broadcasted_iota(jnp.int32, (R, N), 1)
    row_g = rbt * R + jax.lax.broadcasted_iota(jnp.int32, (R, N), 0)
    sim = jnp.where(col == row_g, jnp.float32(-1e9), sim)

    lane_k = jax.lax.broadcasted_iota(jnp.int32, (R, K), 1)
    vals = jnp.zeros((R, K), jnp.float32)
    m = jnp.float32(0)
    for kk in range(K):
        work = sim if kk == 0 else jnp.where(sim < m, sim, -jnp.inf)
        m = jnp.max(work, axis=1, keepdims=True)
        vals = jnp.where(lane_k == kk, m, vals)
    sel = (sim >= m).astype(jnp.float32)

    neigh = jax.lax.dot_general(
        sel, nodes_all, (((1,), (0,)), ((), ())),
        preferred_element_type=jnp.float32) * jnp.float32(1.0 / K)

    msgs = jax.lax.dot_general(
        neigh, wmsg_ref[...], (((1,), (1,)), ((), ())),
        preferred_element_type=jnp.float32) + bmsg_ref[...]
    msgs = jnp.maximum(msgs, 0.0)
    updated = rows + msgs

    out = jax.lax.dot_general(
        updated, wout_ref[...], (((1,), (1,)), ((), ())),
        preferred_element_type=jnp.float32) + bout_ref[...]
    return out, vals


def _fused_kernel(rf_ref, wnode_ref, bnode_ref, wmsg_ref, bmsg_ref,
                  wout_ref, bout_ref, out_ref, vals_ref, nodes_vmem,
                  *, R, K, N, T):
    rb = pl.program_id(1)

    @pl.when(rb == 0)
    def _():
        rf = rf_ref[0]
        nodes_vmem[...] = jax.lax.dot_general(
            rf, wnode_ref[...], (((1,), (1,)), ((), ())),
            preferred_element_type=jnp.float32) + bnode_ref[...]

    for t in range(T):
        out_t, vals_t = _subtile(nodes_vmem, wmsg_ref, bmsg_ref,
                                 wout_ref, bout_ref, rb * T + t, R, K, N)
        out_ref[0, t * R:(t + 1) * R, :] = out_t
        vals_ref[0, t * R:(t + 1) * R, :] = vals_t


def kernel(region_features, W_node, b_node, W_msg, b_msg, W_out, b_out):
    B, N, D = region_features.shape
    H = W_node.shape[0]
    K = min(6, N - 1)
    R = 128
    T = 2

    out, vals = pl.pallas_call(
        functools.partial(_fused_kernel, R=R, K=K, N=N, T=T),
        grid=(B, N // (R * T)),
        in_specs=[
            pl.BlockSpec((1, N, D), lambda b, rb: (b, 0, 0)),
            pl.BlockSpec((H, D), lambda b, rb: (0, 0)),
            pl.BlockSpec((1, H), lambda b, rb: (0, 0)),
            pl.BlockSpec((H, H), lambda b, rb: (0, 0)),
            pl.BlockSpec((1, H), lambda b, rb: (0, 0)),
            pl.BlockSpec((D, H), lambda b, rb: (0, 0)),
            pl.BlockSpec((1, D), lambda b, rb: (0, 0)),
        ],
        out_specs=[
            pl.BlockSpec((1, R * T, D), lambda b, rb: (b, rb, 0)),
            pl.BlockSpec((1, R * T, K), lambda b, rb: (b, rb, 0)),
        ],
        out_shape=[
            jax.ShapeDtypeStruct((B, N, D), jnp.float32),
            jax.ShapeDtypeStruct((B, N, K), jnp.float32),
        ],
        scratch_shapes=[pltpu.VMEM((N, H), jnp.float32)],
        compiler_params=pltpu.CompilerParams(
            dimension_semantics=("arbitrary", "arbitrary")),
    )(region_features, W_node, b_node.reshape(1, H), W_msg,
      b_msg.reshape(1, H), W_out, b_out.reshape(1, D))

    return (out, vals)
